# baseline (speedup 1.0000x reference)
import functools

import jax
import jax.numpy as jnp
from jax import lax
from jax.experimental import pallas as pl
from jax.experimental.pallas import tpu as pltpu
from jax.experimental.pallas import tpu_sc as plsc

B, S, H = 4096, 200, 128
NCP, NSB = 8, 3
NM = NCP * NSB
TBL = S * NM
NTOK = B * S
NW = 32
TOK_PER_W = NTOK // NW
B_PER_W = B // NW
CHUNK = 128
NCHUNK = TOK_PER_W // CHUNK
LANES = 16


def _table_body(char_ref, seq_ref, sbnd_ref, table_ref):
    r_c = lax.broadcasted_iota(jnp.int32, (NM, NCP), 0)
    oh_c = (r_c // NSB == lax.broadcasted_iota(jnp.int32, (NM, NCP), 1)).astype(jnp.float32)
    r_k = lax.broadcasted_iota(jnp.int32, (NM, NSB), 0)
    oh_k = (r_k % NSB == lax.broadcasted_iota(jnp.int32, (NM, NSB), 1)).astype(jnp.float32)
    hi = lax.Precision.HIGHEST
    comb = jnp.dot(oh_c, char_ref[...], preferred_element_type=jnp.float32, precision=hi) + jnp.dot(
        oh_k, sbnd_ref[...], preferred_element_type=jnp.float32, precision=hi
    )
    table_ref[...] = seq_ref[...][:, None, :] + comb[None, :, :]


_sc_mesh = plsc.VectorSubcoreMesh(core_axis_name="c", subcore_axis_name="s")


@functools.partial(
    pl.kernel,
    mesh=_sc_mesh,
    out_type=jax.ShapeDtypeStruct((NTOK, H), jnp.float32),
    scratch_types=[
        pltpu.VMEM_SHARED((TBL, H), jnp.float32),
        pltpu.VMEM((TOK_PER_W,), jnp.int32),
        pltpu.VMEM((TOK_PER_W,), jnp.int32),
        pltpu.VMEM((4, CHUNK), jnp.int32),
        pltpu.VMEM((CHUNK, H), jnp.float32),
        pltpu.VMEM((CHUNK, H), jnp.float32),
        pltpu.SemaphoreType.DMA,
        pltpu.SemaphoreType.DMA,
        pltpu.SemaphoreType.DMA,
        pltpu.SemaphoreType.DMA,
    ],
)
def _sc_gather(table_hbm, cp_hbm, sb_hbm, out_hbm, table_sp, cp_v, sb_v, idx_v, buf0, buf1, sem0, sem1, isem0, isem1):
    sid = lax.axis_index("s")
    wid = sid * 2 + lax.axis_index("c")
    base = wid * TOK_PER_W

    cp_dma = pltpu.async_copy(cp_hbm.at[pl.ds(base, TOK_PER_W)], cp_v, isem0)
    sb_dma = pltpu.async_copy(sb_hbm.at[pl.ds(base, TOK_PER_W)], sb_v, isem1)

    @pl.when(sid == 0)
    def _():
        pltpu.sync_copy(table_hbm, table_sp)

    cp_dma.wait()
    sb_dma.wait()
    plsc.subcore_barrier()

    iota16 = lax.broadcasted_iota(jnp.int32, (LANES,), 0)

    def fill_idx_row(j):
        for k in range(CHUNK // LANES):
            off = j * CHUNK + k * LANES
            s = (off + iota16) % S
            cp16 = cp_v[pl.ds(off, LANES)]
            sb16 = sb_v[pl.ds(off, LANES)]
            idx_v[j % 4, pl.ds(k * LANES, LANES)] = s * NM + cp16 * NSB + sb16

    fill_idx_row(0)
    fill_idx_row(1)
    pltpu.async_copy(table_sp.at[idx_v.at[0]], buf0, sem0)

    def step(i, carry):
        j0 = 2 * i
        pltpu.make_async_copy(out_hbm.at[pl.ds(0, CHUNK)], buf0, sem0).wait()
        pltpu.async_copy(table_sp.at[idx_v.at[(j0 + 1) % 4]], buf1, sem1)

        @pl.when(j0 + 2 < NCHUNK)
        def _():
            fill_idx_row(j0 + 2)

        pltpu.sync_copy(buf0, out_hbm.at[pl.ds(base + j0 * CHUNK, CHUNK)])
        pltpu.make_async_copy(out_hbm.at[pl.ds(0, CHUNK)], buf1, sem1).wait()

        @pl.when(j0 + 2 < NCHUNK)
        def _():
            pltpu.async_copy(table_sp.at[idx_v.at[(j0 + 2) % 4]], buf0, sem0)

        @pl.when(j0 + 3 < NCHUNK)
        def _():
            fill_idx_row(j0 + 3)

        pltpu.sync_copy(buf1, out_hbm.at[pl.ds(base + (j0 + 1) * CHUNK, CHUNK)])
        return carry

    lax.fori_loop(0, NCHUNK // 2, step, 0)


def kernel(input_ids, char_positions, sentence_boundaries, char_pos_embed, seq_pos_embed, sentence_boundary_embed):
    del input_ids
    table3 = pl.pallas_call(
        _table_body,
        out_shape=jax.ShapeDtypeStruct((S, NM, H), jnp.float32),
    )(char_pos_embed, seq_pos_embed, sentence_boundary_embed)

    out = _sc_gather(
        table3.reshape(TBL, H),
        char_positions.astype(jnp.int32).reshape(NTOK),
        sentence_boundaries.astype(jnp.int32).reshape(NTOK),
    )
    return out.reshape(B, S, H)


# speedup vs baseline: 1.0328x; 1.0328x over previous
import functools

import jax
import jax.numpy as jnp
from jax import lax
from jax.experimental import pallas as pl
from jax.experimental.pallas import tpu as pltpu
from jax.experimental.pallas import tpu_sc as plsc

B, S, H = 4096, 200, 128
NCP, NSB = 8, 3
NM = NCP * NSB
TBL = S * NM
NTOK = B * S
NW = 32
TOK_PER_W = NTOK // NW
B_PER_W = B // NW
CHUNK = 128
NCHUNK = TOK_PER_W // CHUNK
LANES = 16


def _table_body(char_ref, seq_ref, sbnd_ref, table_ref):
    r_c = lax.broadcasted_iota(jnp.int32, (NM, NCP), 0)
    oh_c = (r_c // NSB == lax.broadcasted_iota(jnp.int32, (NM, NCP), 1)).astype(jnp.float32)
    r_k = lax.broadcasted_iota(jnp.int32, (NM, NSB), 0)
    oh_k = (r_k % NSB == lax.broadcasted_iota(jnp.int32, (NM, NSB), 1)).astype(jnp.float32)
    hi = lax.Precision.HIGHEST
    comb = jnp.dot(oh_c, char_ref[...], preferred_element_type=jnp.float32, precision=hi) + jnp.dot(
        oh_k, sbnd_ref[...], preferred_element_type=jnp.float32, precision=hi
    )
    table_ref[...] = seq_ref[...][:, None, :] + comb[None, :, :]


_sc_mesh = plsc.VectorSubcoreMesh(core_axis_name="c", subcore_axis_name="s")


@functools.partial(
    pl.kernel,
    mesh=_sc_mesh,
    out_type=jax.ShapeDtypeStruct((NTOK, H), jnp.float32),
    scratch_types=[
        pltpu.VMEM_SHARED((TBL, H), jnp.float32),
        pltpu.VMEM((TOK_PER_W,), jnp.int32),
        pltpu.VMEM((4, CHUNK), jnp.int32),
        pltpu.VMEM((CHUNK, H), jnp.float32),
        pltpu.VMEM((CHUNK, H), jnp.float32),
        pltpu.SemaphoreType.DMA,
        pltpu.SemaphoreType.DMA,
        pltpu.SemaphoreType.DMA,
    ],
)
def _sc_gather(table_hbm, m24_hbm, out_hbm, table_sp, m24_v, idx_v, buf0, buf1, sem0, sem1, isem0):
    sid = lax.axis_index("s")
    wid = sid * 2 + lax.axis_index("c")
    base = wid * TOK_PER_W

    m24_dma = pltpu.async_copy(m24_hbm.at[pl.ds(base, TOK_PER_W)], m24_v, isem0)

    @pl.when(sid == 0)
    def _():
        pltpu.sync_copy(table_hbm, table_sp)

    m24_dma.wait()
    plsc.subcore_barrier()

    iota16 = lax.broadcasted_iota(jnp.int32, (LANES,), 0)

    def fill_idx_row(j):
        for k in range(CHUNK // LANES):
            off = j * CHUNK + k * LANES
            s = (off + iota16) % S
            idx_v[j % 4, pl.ds(k * LANES, LANES)] = s * NM + m24_v[pl.ds(off, LANES)]

    fill_idx_row(0)
    fill_idx_row(1)
    pltpu.async_copy(table_sp.at[idx_v.at[0]], buf0, sem0)

    def step(i, carry):
        j0 = 2 * i
        pltpu.make_async_copy(out_hbm.at[pl.ds(0, CHUNK)], buf0, sem0).wait()
        pltpu.async_copy(table_sp.at[idx_v.at[(j0 + 1) % 4]], buf1, sem1)

        @pl.when(j0 + 2 < NCHUNK)
        def _():
            fill_idx_row(j0 + 2)

        pltpu.sync_copy(buf0, out_hbm.at[pl.ds(base + j0 * CHUNK, CHUNK)])
        pltpu.make_async_copy(out_hbm.at[pl.ds(0, CHUNK)], buf1, sem1).wait()

        @pl.when(j0 + 2 < NCHUNK)
        def _():
            pltpu.async_copy(table_sp.at[idx_v.at[(j0 + 2) % 4]], buf0, sem0)

        @pl.when(j0 + 3 < NCHUNK)
        def _():
            fill_idx_row(j0 + 3)

        pltpu.sync_copy(buf1, out_hbm.at[pl.ds(base + (j0 + 1) * CHUNK, CHUNK)])
        return carry

    lax.fori_loop(0, NCHUNK // 2, step, 0)


def kernel(input_ids, char_positions, sentence_boundaries, char_pos_embed, seq_pos_embed, sentence_boundary_embed):
    del input_ids
    table3 = pl.pallas_call(
        _table_body,
        out_shape=jax.ShapeDtypeStruct((S, NM, H), jnp.float32),
    )(char_pos_embed, seq_pos_embed, sentence_boundary_embed)

    m24 = (char_positions.astype(jnp.int32) * NSB + sentence_boundaries.astype(jnp.int32)).reshape(NTOK)
    out = _sc_gather(table3.reshape(TBL, H), m24)
    return out.reshape(B, S, H)
